# baseline (speedup 1.0000x reference)
import jax
import jax.numpy as jnp
from jax.experimental import pallas as pl
from jax.experimental.pallas import tpu as pltpu

VOCAB_ = 1000000
DIMS_ = 64
FOLD = 8
ROWS = VOCAB_ // FOLD
LANES = FOLD * DIMS_
BLK8 = 8192
NB = (ROWS + BLK8 - 1) // BLK8
NEG_INF = float("-inf")


def _body(idx_ref, table_ref, w_ref, b_ref, out_ref, z_ref, m_ref, s_ref):
    t = pl.program_id(0)

    @pl.when(t == 0)
    def _init():
        m_ref[...] = jnp.full_like(m_ref, NEG_INF)
        s_ref[...] = jnp.zeros_like(s_ref)

    @pl.when(t < NB)
    def _compute():
        r = idx_ref[0] % 8
        e = table_ref[pl.ds(r, 1), :]
        e_tile = jnp.broadcast_to(
            jnp.concatenate([e] * FOLD, axis=1), (FOLD, LANES))
        lane = jax.lax.broadcasted_iota(jnp.int32, (FOLD, LANES), 1)
        sub = jax.lax.broadcasted_iota(jnp.int32, (FOLD, LANES), 0)
        e8 = jnp.where(lane // DIMS_ == sub, e_tile, 0.0)
        z = jax.lax.dot_general(
            e8, w_ref[...], (((1,), (1,)), ((), ())),
            preferred_element_type=jnp.float32)
        z = z + b_ref[...]
        z_ref[:, pl.ds(t * BLK8, BLK8)] = z
        k = jax.lax.broadcasted_iota(jnp.int32, (FOLD, BLK8), 1)
        p = jax.lax.broadcasted_iota(jnp.int32, (FOLD, BLK8), 0)
        gidx = FOLD * (t * BLK8 + k) + p
        zm = jnp.where(gidx < VOCAB_, z, NEG_INF)
        bm = jnp.max(zm, keepdims=True)
        new_m = jnp.maximum(m_ref[...], bm)
        s_ref[...] = s_ref[...] * jnp.exp(m_ref[...] - new_m) + jnp.sum(
            jnp.exp(zm - new_m), keepdims=True)
        m_ref[...] = new_m

    @pl.when(t >= NB)
    def _write():
        j = t - NB
        lse = m_ref[...] + jnp.log(s_ref[...])
        out_ref[...] = z_ref[:, pl.ds(j * BLK8, BLK8)] - lse


@jax.jit
def _run(inputs, table, W8, b8):
    grid_spec = pltpu.PrefetchScalarGridSpec(
        num_scalar_prefetch=1,
        grid=(2 * NB,),
        in_specs=[
            pl.BlockSpec((8, DIMS_), lambda t, idx: (idx[0] // 8, 0)),
            pl.BlockSpec((BLK8, LANES),
                         lambda t, idx: (jnp.minimum(t, NB - 1), 0)),
            pl.BlockSpec((FOLD, BLK8),
                         lambda t, idx: (0, jnp.minimum(t, NB - 1))),
        ],
        out_specs=pl.BlockSpec(
            (FOLD, BLK8), lambda t, idx: (0, jnp.where(t < NB, 0, t - NB))),
        scratch_shapes=[
            pltpu.VMEM((FOLD, NB * BLK8), jnp.float32),
            pltpu.VMEM((1, 1), jnp.float32),
            pltpu.VMEM((1, 1), jnp.float32),
        ],
    )
    return pl.pallas_call(
        _body,
        grid_spec=grid_spec,
        out_shape=jax.ShapeDtypeStruct((FOLD, ROWS), jnp.float32),
        compiler_params=pltpu.CompilerParams(
            dimension_semantics=("arbitrary",),
        ),
    )(inputs, table, W8, b8)


def kernel(inputs, table, W, b):
    idx = inputs.astype(jnp.int32)
    W8 = W.reshape(ROWS, LANES)
    b8 = b.reshape(ROWS, FOLD).T
    out8 = _run(idx, table, W8, b8)
    return out8.T.reshape(1, VOCAB_)
